# baseline (speedup 1.0000x reference)
import jax
import jax.numpy as jnp
import numpy as np
from jax import lax
from jax.experimental import pallas as pl
from jax.experimental.pallas import tpu as pltpu
from jax.experimental.pallas import tpu_sc as plsc

VOCAB = 100000
DIM = 128
B = 16384
L = 200
NC, NS = 2, 16
NW = NC * NS
BPW = B // NW
NCHUNK = 4
CR = BPW // NCHUNK
LH1, LH2 = 104, 96
LPAD = 208


def _body(focus_hbm, ctx_hbm, emb_hbm, out_hbm,
          fidx, frows, cidx, bufA, bufB, obufA, obufB,
          tbufA, tbufB, tbufC, tbufD,
          semF, semA, semB, semOA, semOB):
    wid = lax.axis_index("s") * NC + lax.axis_index("c")
    wbase = wid * BPW
    pltpu.sync_copy(focus_hbm.at[pl.ds(wbase, BPW)], fidx)
    col0 = lax.iota(jnp.int32, 16) * 24

    def fire(r, buf, sem):
        pltpu.make_async_copy(
            emb_hbm.at[cidx.at[r, pl.ds(0, LH1)]], buf.at[pl.ds(0, LH1)],
            sem).start()
        pltpu.make_async_copy(
            emb_hbm.at[cidx.at[r, pl.ds(LH1, LH2)]], buf.at[pl.ds(LH1, LH2)],
            sem).start()

    def drain(r, buf, sem):
        pltpu.make_async_copy(
            emb_hbm.at[cidx.at[r, pl.ds(0, LH1)]], buf.at[pl.ds(0, LH1)],
            sem).wait()
        pltpu.make_async_copy(
            emb_hbm.at[cidx.at[r, pl.ds(LH1, LH2)]], buf.at[pl.ds(LH1, LH2)],
            sem).wait()

    def compute_row(buf, r, obuf, tbufA, tbufB):
        fp = [frows[r, pl.ds(j * 32, 32)] for j in range(DIM // 32)]

        def do_group(gbase, tbuf):
            for k in range(16):
                us = []
                for j in range(DIM // 32):
                    p = buf[gbase + k, pl.ds(j * 32, 32)] * fp[j]
                    ua, ub = plsc.unpack(p, format=plsc.PackFormat.INTERLEAVED)
                    us.append(ua)
                    us.append(ub)
                tbuf[pl.ds(k * 24, 16)] = (us[0] + us[1]) + (us[2] + us[3])
                plsc.addupdate(tbuf.at[pl.ds(k * 24, 16)],
                               (us[4] + us[5]) + (us[6] + us[7]))

        def red_group(gbase, tbuf):
            cols = [col0 + j for j in range(4)]
            tots = [plsc.load_gather(tbuf, [c]) for c in cols]
            for j in range(4, 16):
                tots[j % 4] = tots[j % 4] + plsc.load_gather(
                    tbuf, [cols[j % 4] + (j - j % 4)])
            obuf[pl.ds(gbase, 16)] = (tots[0] + tots[1]) + (tots[2] + tots[3])

        def gloop(i, _):
            do_group(32 * i, tbufA)
            do_group(32 * i + 16, tbufB)
            red_group(32 * i, tbufA)
            red_group(32 * i + 16, tbufB)
            return _

        lax.fori_loop(0, (L // 16) // 2, gloop, None)
        do_group((L // 16) * 16, tbufA)
        red_group((L // 16) * 16, tbufA)

    def out_wait(c, r, obuf, sem):
        @pl.when((c > 0) | (r >= 2))
        def _():
            pltpu.make_async_copy(
                obuf.at[pl.ds(0, L)], out_hbm.at[0], sem).wait()

    def out_fire(c, r, obuf, sem):
        pltpu.make_async_copy(
            obuf.at[pl.ds(0, L)], out_hbm.at[wbase + c * CR + r], sem).start()

    def chunk(c, _):
        pltpu.sync_copy(ctx_hbm.at[pl.ds(wbase + c * CR, CR)], cidx)
        pltpu.async_copy(
            emb_hbm.at[fidx.at[pl.ds(c * CR, CR)]], frows, semF).wait()
        fire(0, bufA, semA)

        def step(g, _):
            r0 = 2 * g
            r1 = r0 + 1
            fire(r1, bufB, semB)
            drain(r0, bufA, semA)
            out_wait(c, r0, obufA, semOA)
            compute_row(bufA, r0, obufA, tbufA, tbufB)
            out_fire(c, r0, obufA, semOA)

            @pl.when(r0 + 2 < CR)
            def _():
                fire(r0 + 2, bufA, semA)

            drain(r1, bufB, semB)
            out_wait(c, r1, obufB, semOB)
            compute_row(bufB, r1, obufB, tbufC, tbufD)
            out_fire(c, r1, obufB, semOB)
            return _

        lax.fori_loop(0, CR // 2, step, None)
        return _

    lax.fori_loop(0, NCHUNK, chunk, None)
    pltpu.make_async_copy(
        obufA.at[pl.ds(0, L)], out_hbm.at[0], semOA).wait()
    pltpu.make_async_copy(
        obufB.at[pl.ds(0, L)], out_hbm.at[0], semOB).wait()


_sc_call = pl.kernel(
    _body,
    out_type=jax.ShapeDtypeStruct((B, L), jnp.float32),
    mesh=plsc.VectorSubcoreMesh(
        core_axis_name="c", subcore_axis_name="s",
        num_cores=NC, num_subcores=NS),
    scratch_types=[
        pltpu.VMEM((BPW,), jnp.int32),
        pltpu.VMEM((CR, DIM), jnp.bfloat16),
        pltpu.VMEM((CR, L), jnp.int32),
        pltpu.VMEM((LPAD, DIM), jnp.bfloat16),
        pltpu.VMEM((LPAD, DIM), jnp.bfloat16),
        pltpu.VMEM((LPAD,), jnp.float32),
        pltpu.VMEM((LPAD,), jnp.float32),
        pltpu.VMEM((384,), jnp.float32),
        pltpu.VMEM((384,), jnp.float32),
        pltpu.VMEM((384,), jnp.float32),
        pltpu.VMEM((384,), jnp.float32),
        pltpu.SemaphoreType.DMA,
        pltpu.SemaphoreType.DMA,
        pltpu.SemaphoreType.DMA,
        pltpu.SemaphoreType.DMA,
        pltpu.SemaphoreType.DMA,
    ],
    compiler_params=pltpu.CompilerParams(
        needs_layout_passes=False, use_tc_tiling_on_sc=False),
)


def kernel(focus_item_batch, context_items_batch, embeddings):
    return _sc_call(focus_item_batch.reshape(B), context_items_batch,
                    embeddings.astype(jnp.bfloat16))


# speedup vs baseline: 1.0558x; 1.0558x over previous
import jax
import jax.numpy as jnp
import numpy as np
from jax import lax
from jax.experimental import pallas as pl
from jax.experimental.pallas import tpu as pltpu
from jax.experimental.pallas import tpu_sc as plsc

VOCAB = 100000
DIM = 128
B = 16384
L = 200
NC, NS = 2, 16
NW = NC * NS
BPW = B // NW
NCHUNK = 4
CR = BPW // NCHUNK
LH1, LH2 = 104, 96
LPAD = 208


def _body(focus_hbm, ctx_hbm, emb_hbm, out_hbm,
          fidx, frows, cidx, bufA, bufB, obufA, obufB,
          tbufA, tbufB, tbufC, tbufD,
          semF, semA, semB, semOA, semOB):
    wid = lax.axis_index("s") * NC + lax.axis_index("c")
    wbase = wid * BPW
    pltpu.sync_copy(focus_hbm.at[pl.ds(wbase, BPW)], fidx)
    col0 = lax.iota(jnp.int32, 16) * 24

    def fire(r, buf, sem):
        pltpu.make_async_copy(
            emb_hbm.at[cidx.at[r, pl.ds(0, LH1)]], buf.at[pl.ds(0, LH1)],
            sem).start()
        pltpu.make_async_copy(
            emb_hbm.at[cidx.at[r, pl.ds(LH1, LH2)]], buf.at[pl.ds(LH1, LH2)],
            sem).start()

    def drain(r, buf, sem):
        pltpu.make_async_copy(
            emb_hbm.at[cidx.at[r, pl.ds(0, LH1)]], buf.at[pl.ds(0, LH1)],
            sem).wait()
        pltpu.make_async_copy(
            emb_hbm.at[cidx.at[r, pl.ds(LH1, LH2)]], buf.at[pl.ds(LH1, LH2)],
            sem).wait()

    def compute_row(buf, r, obuf, tbufA, tbufB):
        fp = [frows[r, pl.ds(j * 32, 32)] for j in range(DIM // 32)]

        def do_group(gbase, tbuf):
            for k in range(16):
                us = []
                for j in range(DIM // 32):
                    p = buf[gbase + k, pl.ds(j * 32, 32)] * fp[j]
                    ua, ub = plsc.unpack(p, format=plsc.PackFormat.INTERLEAVED)
                    us.append(ua)
                    us.append(ub)
                a = (us[0] + us[1]) + (us[2] + us[3])
                b = (us[4] + us[5]) + (us[6] + us[7])
                tbuf[pl.ds(k * 24, 16)] = a + b

        def red_group(gbase, tbuf):
            cols = [col0 + j for j in range(4)]
            tots = [plsc.load_gather(tbuf, [c]) for c in cols]
            for j in range(4, 16):
                tots[j % 4] = tots[j % 4] + plsc.load_gather(
                    tbuf, [cols[j % 4] + (j - j % 4)])
            obuf[pl.ds(gbase, 16)] = (tots[0] + tots[1]) + (tots[2] + tots[3])

        def gloop(i, _):
            do_group(32 * i, tbufA)
            do_group(32 * i + 16, tbufB)
            red_group(32 * i, tbufA)
            red_group(32 * i + 16, tbufB)
            return _

        lax.fori_loop(0, (L // 16) // 2, gloop, None)
        do_group((L // 16) * 16, tbufA)
        red_group((L // 16) * 16, tbufA)

    def out_wait(c, r, obuf, sem):
        @pl.when((c > 0) | (r >= 2))
        def _():
            pltpu.make_async_copy(
                obuf.at[pl.ds(0, L)], out_hbm.at[0], sem).wait()

    def out_fire(c, r, obuf, sem):
        pltpu.make_async_copy(
            obuf.at[pl.ds(0, L)], out_hbm.at[wbase + c * CR + r], sem).start()

    def chunk(c, _):
        pltpu.sync_copy(ctx_hbm.at[pl.ds(wbase + c * CR, CR)], cidx)
        pltpu.async_copy(
            emb_hbm.at[fidx.at[pl.ds(c * CR, CR)]], frows, semF).wait()
        fire(0, bufA, semA)

        def step(g, _):
            r0 = 2 * g
            r1 = r0 + 1
            fire(r1, bufB, semB)
            drain(r0, bufA, semA)
            out_wait(c, r0, obufA, semOA)
            compute_row(bufA, r0, obufA, tbufA, tbufB)
            out_fire(c, r0, obufA, semOA)

            @pl.when(r0 + 2 < CR)
            def _():
                fire(r0 + 2, bufA, semA)

            drain(r1, bufB, semB)
            out_wait(c, r1, obufB, semOB)
            compute_row(bufB, r1, obufB, tbufC, tbufD)
            out_fire(c, r1, obufB, semOB)
            return _

        lax.fori_loop(0, CR // 2, step, None)
        return _

    lax.fori_loop(0, NCHUNK, chunk, None)
    pltpu.make_async_copy(
        obufA.at[pl.ds(0, L)], out_hbm.at[0], semOA).wait()
    pltpu.make_async_copy(
        obufB.at[pl.ds(0, L)], out_hbm.at[0], semOB).wait()


_sc_call = pl.kernel(
    _body,
    out_type=jax.ShapeDtypeStruct((B, L), jnp.float32),
    mesh=plsc.VectorSubcoreMesh(
        core_axis_name="c", subcore_axis_name="s",
        num_cores=NC, num_subcores=NS),
    scratch_types=[
        pltpu.VMEM((BPW,), jnp.int32),
        pltpu.VMEM((CR, DIM), jnp.bfloat16),
        pltpu.VMEM((CR, L), jnp.int32),
        pltpu.VMEM((LPAD, DIM), jnp.bfloat16),
        pltpu.VMEM((LPAD, DIM), jnp.bfloat16),
        pltpu.VMEM((LPAD,), jnp.float32),
        pltpu.VMEM((LPAD,), jnp.float32),
        pltpu.VMEM((384,), jnp.float32),
        pltpu.VMEM((384,), jnp.float32),
        pltpu.VMEM((384,), jnp.float32),
        pltpu.VMEM((384,), jnp.float32),
        pltpu.SemaphoreType.DMA,
        pltpu.SemaphoreType.DMA,
        pltpu.SemaphoreType.DMA,
        pltpu.SemaphoreType.DMA,
        pltpu.SemaphoreType.DMA,
    ],
    compiler_params=pltpu.CompilerParams(
        needs_layout_passes=False, use_tc_tiling_on_sc=False),
)


def kernel(focus_item_batch, context_items_batch, embeddings):
    return _sc_call(focus_item_batch.reshape(B), context_items_batch,
                    embeddings.astype(jnp.bfloat16))
